# Initial kernel scaffold; baseline (speedup 1.0000x reference)
import jax
import jax.numpy as jnp
from jax.experimental import pallas as pl


def kernel(H, Xe, id_Xe, batch_idx, Wm0, bm0, Wu0, bu0, Wm1, bm1, Wu1, bu1, Wm2, bm2, Wu2, bu2, Wmlp, bmlp):
    raise NotImplementedError("write your pallas kernel here")


# speedup vs baseline: 2.4061x; 2.4061x over previous
import functools

import jax
import jax.numpy as jnp
from jax import lax
from jax.experimental import pallas as pl
from jax.experimental.pallas import tpu as pltpu
from jax.experimental.pallas import tpu_sc as plsc

NC = 2
NS = 16
NW = NC * NS


def _z_body(y_ref, w_ref, b_ref, z_ref):
    z_ref[...] = jnp.dot(y_ref[...], w_ref[...],
                         preferred_element_type=jnp.float32) + b_ref[...]


def _z_matmul(y, w, b, block):
    n, d = y.shape
    h = w.shape[1]
    return pl.pallas_call(
        _z_body,
        grid=(n // block,),
        in_specs=[pl.BlockSpec((block, d), lambda i: (i, 0)),
                  pl.BlockSpec((d, h), lambda i: (0, 0)),
                  pl.BlockSpec((1, h), lambda i: (0, 0))],
        out_specs=pl.BlockSpec((block, h), lambda i: (i, 0)),
        out_shape=jax.ShapeDtypeStruct((n, h), jnp.float32),
    )(y, w, b.reshape(1, h))


def _c_body(xe_ref, w0_ref, w1_ref, w2_ref, c0_ref, c1_ref, c2_ref):
    xe = xe_ref[...]
    c0_ref[...] = jnp.dot(xe, w0_ref[...], preferred_element_type=jnp.float32)
    c1_ref[...] = jnp.dot(xe, w1_ref[...], preferred_element_type=jnp.float32)
    c2_ref[...] = jnp.dot(xe, w2_ref[...], preferred_element_type=jnp.float32)


def _c_matmuls(xe, w0, w1, w2, block):
    e, de = xe.shape
    h = w0.shape[1]
    ws = pl.BlockSpec((de, h), lambda i: (0, 0))
    cs = pl.BlockSpec((block, h), lambda i: (i, 0))
    sh = jax.ShapeDtypeStruct((e, h), jnp.float32)
    return pl.pallas_call(
        _c_body,
        grid=(e // block,),
        in_specs=[pl.BlockSpec((block, de), lambda i: (i, 0)), ws, ws, ws],
        out_specs=[cs, cs, cs],
        out_shape=[sh, sh, sh],
    )(xe, w0, w1, w2)


def _upd_body(a0_ref, a1_ref, y_ref, wa_ref, wy_ref, b_ref, o_ref):
    agg = a0_ref[...] + a1_ref[...]
    t = jnp.dot(agg, wa_ref[...], preferred_element_type=jnp.float32)
    t += jnp.dot(y_ref[...], wy_ref[...], preferred_element_type=jnp.float32)
    o_ref[...] = jnp.maximum(t + b_ref[...], 0.0)


def _update(a0, a1, y, wa, wy, b, block):
    n, h = y.shape
    xs = pl.BlockSpec((block, h), lambda i: (i, 0))
    ws = pl.BlockSpec((h, h), lambda i: (0, 0))
    return pl.pallas_call(
        _upd_body,
        grid=(n // block,),
        in_specs=[xs, xs, xs, ws, ws, pl.BlockSpec((1, h), lambda i: (0, 0))],
        out_specs=xs,
        out_shape=jax.ShapeDtypeStruct((n, h), jnp.float32),
    )(a0, a1, y, wa, wy, b.reshape(1, h))


def _pool_body(y_ref, bidx_ref, wm_ref, bm_ref, o_ref):
    i = pl.program_id(0)
    g = o_ref.shape[0]
    t = jnp.dot(y_ref[...], wm_ref[...], preferred_element_type=jnp.float32)
    bidx = bidx_ref[0, 0, :]
    onehot = (bidx[None, :] ==
              lax.broadcasted_iota(jnp.int32, (g, bidx.shape[0]), 0)
              ).astype(jnp.float32)
    part = jnp.dot(onehot, t, preferred_element_type=jnp.float32)

    @pl.when(i == 0)
    def _():
        o_ref[...] = part + bm_ref[...]

    @pl.when(i != 0)
    def _():
        o_ref[...] += part


def _pool(y, batch_idx, wmlp, bmlp, g, block):
    n, h = y.shape
    nb = n // block
    bidx3 = batch_idx.reshape(nb, 1, block)
    return pl.pallas_call(
        _pool_body,
        grid=(nb,),
        in_specs=[pl.BlockSpec((block, h), lambda i: (i, 0)),
                  pl.BlockSpec((1, 1, block), lambda i: (i, 0, 0)),
                  pl.BlockSpec((h, 1), lambda i: (0, 0)),
                  pl.BlockSpec((1, 1), lambda i: (0, 0))],
        out_specs=pl.BlockSpec((g, 1), lambda i: (0, 0)),
        out_shape=jax.ShapeDtypeStruct((g, 1), jnp.float32),
    )(y, bidx3, wmlp, bmlp.reshape(1, 1))


def _edge_kernel(z, c, src, dst, n, e, h):
    epw = e // NW
    ch = 80
    nchunk = epw // ch
    zb = 128
    rows_per_tile = zb * (-(-n // (NS * zb)))
    n2 = NS * rows_per_tile

    mesh = plsc.VectorSubcoreMesh(core_axis_name="c", subcore_axis_name="s",
                                  num_cores=NC, num_subcores=NS)

    @functools.partial(
        pl.kernel,
        mesh=mesh,
        out_type=jax.ShapeDtypeStruct((NC, n2, h), jnp.float32),
        scratch_types=[
            pltpu.VMEM((ch,), jnp.int32),
            pltpu.VMEM((ch,), jnp.int32),
            pltpu.VMEM((ch, h), jnp.float32),
            pltpu.VMEM((ch, h), jnp.float32),
            pltpu.VMEM((zb, h), jnp.float32),
            pltpu.VMEM_SHARED((n2, h), jnp.float32),
            pltpu.SemaphoreType.DMA,
        ],
    )
    def edge(z_hbm, c_hbm, src_hbm, dst_hbm, out_hbm,
             src_v, dst_v, rows_v, cc_v, zero_v, agg_sh, sem):
        ci = lax.axis_index("c")
        si = lax.axis_index("s")
        wid = si * NC + ci

        def zfill(i, carry):
            for j in range(h // 16):
                zero_v[i, pl.ds(j * 16, 16)] = jnp.zeros((16,), jnp.float32)
            return carry
        lax.fori_loop(0, zb, zfill, 0)
        row0 = si * rows_per_tile
        for k in range(rows_per_tile // zb):
            pltpu.sync_copy(zero_v, agg_sh.at[pl.ds(row0 + k * zb, zb)])
        plsc.subcore_barrier()

        base0 = wid * epw

        def chunk(t, carry):
            off = base0 + t * ch
            pltpu.sync_copy(src_hbm.at[pl.ds(off, ch)], src_v)
            pltpu.sync_copy(dst_hbm.at[pl.ds(off, ch)], dst_v)
            pltpu.async_copy(z_hbm.at[src_v], rows_v, sem).wait()
            pltpu.sync_copy(c_hbm.at[pl.ds(off, ch)], cc_v)

            def erow(r, icarry):
                for j in range(h // 16):
                    sl = pl.ds(j * 16, 16)
                    rows_v[r, sl] = jnp.maximum(rows_v[r, sl] + cc_v[r, sl],
                                                0.0)
                return icarry
            lax.fori_loop(0, ch, erow, 0)
            pltpu.sync_copy(rows_v, agg_sh.at[dst_v], add=True)
            return carry
        lax.fori_loop(0, nchunk, chunk, 0)

        plsc.subcore_barrier()
        for k in range(rows_per_tile // zb):
            sl = pl.ds(row0 + k * zb, zb)
            pltpu.sync_copy(agg_sh.at[sl], out_hbm.at[ci, sl])

    return edge(z, c, src, dst)


def kernel(H, Xe, id_Xe, batch_idx,
           Wm0, bm0, Wu0, bu0,
           Wm1, bm1, Wu1, bu1,
           Wm2, bm2, Wu2, bu2,
           Wmlp, bmlp):
    n, d = H.shape
    e, de = Xe.shape
    hid = Wm0.shape[1]
    g = 64
    src = id_Xe[0]
    dst = id_Xe[1]

    c0, c1, c2 = _c_matmuls(Xe, Wm0[d:], Wm1[hid:], Wm2[hid:], block=2000)

    y = H
    for wm, bm, wu, bu, c in ((Wm0, bm0, Wu0, bu0, c0),
                              (Wm1, bm1, Wu1, bu1, c1),
                              (Wm2, bm2, Wu2, bu2, c2)):
        din = wm.shape[0] - de
        z = _z_matmul(y, wm[:din], bm, block=1000)
        parts = _edge_kernel(z, c, src, dst, n, e, hid)
        y = _update(parts[0], parts[1], y, wu[:hid], wu[hid:], bu, block=1000)

    return _pool(y, batch_idx, Wmlp, bmlp, g, block=1000)


# speedup vs baseline: 4.3844x; 1.8222x over previous
import functools

import jax
import jax.numpy as jnp
from jax import lax
from jax.experimental import pallas as pl
from jax.experimental.pallas import tpu as pltpu
from jax.experimental.pallas import tpu_sc as plsc

NC = 2
NS = 16
NW = NC * NS


def _z_body(y_ref, w_ref, b_ref, z_ref):
    z_ref[...] = jnp.dot(y_ref[...], w_ref[...],
                         preferred_element_type=jnp.float32) + b_ref[...]


def _z_matmul(y, w, b, block):
    n, d = y.shape
    h = w.shape[1]
    return pl.pallas_call(
        _z_body,
        grid=(n // block,),
        in_specs=[pl.BlockSpec((block, d), lambda i: (i, 0)),
                  pl.BlockSpec((d, h), lambda i: (0, 0)),
                  pl.BlockSpec((1, h), lambda i: (0, 0))],
        out_specs=pl.BlockSpec((block, h), lambda i: (i, 0)),
        out_shape=jax.ShapeDtypeStruct((n, h), jnp.float32),
    )(y, w, b.reshape(1, h))


def _c_body(xe_ref, w0_ref, w1_ref, w2_ref, c0_ref, c1_ref, c2_ref):
    xe = xe_ref[...]
    c0_ref[...] = jnp.dot(xe, w0_ref[...], preferred_element_type=jnp.float32)
    c1_ref[...] = jnp.dot(xe, w1_ref[...], preferred_element_type=jnp.float32)
    c2_ref[...] = jnp.dot(xe, w2_ref[...], preferred_element_type=jnp.float32)


def _c_matmuls(xe, w0, w1, w2, block):
    e, de = xe.shape
    h = w0.shape[1]
    ws = pl.BlockSpec((de, h), lambda i: (0, 0))
    cs = pl.BlockSpec((block, h), lambda i: (i, 0))
    sh = jax.ShapeDtypeStruct((e, h), jnp.float32)
    return pl.pallas_call(
        _c_body,
        grid=(e // block,),
        in_specs=[pl.BlockSpec((block, de), lambda i: (i, 0)), ws, ws, ws],
        out_specs=[cs, cs, cs],
        out_shape=[sh, sh, sh],
    )(xe, w0, w1, w2)


def _upd_body(a0_ref, a1_ref, y_ref, wa_ref, wy_ref, b_ref, o_ref):
    agg = a0_ref[...] + a1_ref[...]
    t = jnp.dot(agg, wa_ref[...], preferred_element_type=jnp.float32)
    t += jnp.dot(y_ref[...], wy_ref[...], preferred_element_type=jnp.float32)
    o_ref[...] = jnp.maximum(t + b_ref[...], 0.0)


def _update(a0, a1, y, wa, wy, b, block):
    n, h = y.shape
    xs = pl.BlockSpec((block, h), lambda i: (i, 0))
    ws = pl.BlockSpec((h, h), lambda i: (0, 0))
    return pl.pallas_call(
        _upd_body,
        grid=(n // block,),
        in_specs=[xs, xs, xs, ws, ws, pl.BlockSpec((1, h), lambda i: (0, 0))],
        out_specs=xs,
        out_shape=jax.ShapeDtypeStruct((n, h), jnp.float32),
    )(a0, a1, y, wa, wy, b.reshape(1, h))


def _pool_body(y_ref, bidx_ref, wm_ref, bm_ref, o_ref):
    i = pl.program_id(0)
    g = o_ref.shape[0]
    t = jnp.dot(y_ref[...], wm_ref[...], preferred_element_type=jnp.float32)
    bidx = bidx_ref[0, 0, :]
    onehot = (bidx[None, :] ==
              lax.broadcasted_iota(jnp.int32, (g, bidx.shape[0]), 0)
              ).astype(jnp.float32)
    part = jnp.dot(onehot, t, preferred_element_type=jnp.float32)

    @pl.when(i == 0)
    def _():
        o_ref[...] = part + bm_ref[...]

    @pl.when(i != 0)
    def _():
        o_ref[...] += part


def _pool(y, batch_idx, wmlp, bmlp, g, block):
    n, h = y.shape
    nb = n // block
    bidx3 = batch_idx.reshape(nb, 1, block)
    return pl.pallas_call(
        _pool_body,
        grid=(nb,),
        in_specs=[pl.BlockSpec((block, h), lambda i: (i, 0)),
                  pl.BlockSpec((1, 1, block), lambda i: (i, 0, 0)),
                  pl.BlockSpec((h, 1), lambda i: (0, 0)),
                  pl.BlockSpec((1, 1), lambda i: (0, 0))],
        out_specs=pl.BlockSpec((g, 1), lambda i: (0, 0)),
        out_shape=jax.ShapeDtypeStruct((g, 1), jnp.float32),
    )(y, bidx3, wmlp, bmlp.reshape(1, 1))


def _edge_kernel(z, c, src, dst, n, e, h):
    epw = e // NW
    ch = 80
    nchunk = epw // ch
    assert nchunk % 2 == 1 and nchunk >= 5
    rows_per_tile = ch * (-(-n // (NS * ch)))
    n2 = NS * rows_per_tile

    mesh = plsc.VectorSubcoreMesh(core_axis_name="c", subcore_axis_name="s",
                                  num_cores=NC, num_subcores=NS)

    @functools.partial(
        pl.kernel,
        mesh=mesh,
        out_type=jax.ShapeDtypeStruct((NC, n2, h), jnp.float32),
        scratch_types=[
            pltpu.VMEM((ch,), jnp.int32),
            pltpu.VMEM((ch,), jnp.int32),
            pltpu.VMEM((ch,), jnp.int32),
            pltpu.VMEM((ch,), jnp.int32),
            pltpu.VMEM((ch, h), jnp.float32),
            pltpu.VMEM((ch, h), jnp.float32),
            pltpu.VMEM((ch, h), jnp.float32),
            pltpu.VMEM((ch, h), jnp.float32),
            pltpu.VMEM_SHARED((n2, h), jnp.float32),
        ] + [pltpu.SemaphoreType.DMA] * 10,
    )
    def edge(z_hbm, c_hbm, src_hbm, dst_hbm, out_hbm,
             srcs0, srcs1, dsts0, dsts1, rows0, rows1, cc0, cc1, agg_sh,
             sg0, sg1, sc0, sc1, ss0, ss1, sis0, sis1, sid0, sid1):
        ci = lax.axis_index("c")
        si = lax.axis_index("s")
        wid = si * NC + ci
        base0 = wid * epw
        srcs = (srcs0, srcs1)
        dsts = (dsts0, dsts1)
        rows = (rows0, rows1)
        cc = (cc0, cc1)
        sg = (sg0, sg1)
        sc = (sc0, sc1)
        ss = (ss0, ss1)
        sis = (sis0, sis1)
        sid = (sid0, sid1)

        def zfill(i, carry):
            for j in range(h // 16):
                rows0[i, pl.ds(j * 16, 16)] = jnp.zeros((16,), jnp.float32)
            return carry
        lax.fori_loop(0, ch, zfill, 0)
        row0 = si * rows_per_tile
        for k in range(rows_per_tile // ch):
            pltpu.sync_copy(rows0, agg_sh.at[pl.ds(row0 + k * ch, ch)])
        plsc.subcore_barrier()

        def issue_src(t, b):
            pltpu.async_copy(src_hbm.at[pl.ds(base0 + t * ch, ch)],
                             srcs[b], sis[b])

        def wait_src(t, b):
            pltpu.make_async_copy(src_hbm.at[pl.ds(base0 + t * ch, ch)],
                                  srcs[b], sis[b]).wait()

        def issue_dst(t, b):
            pltpu.async_copy(dst_hbm.at[pl.ds(base0 + t * ch, ch)],
                             dsts[b], sid[b])

        def wait_dst(t, b):
            pltpu.make_async_copy(dst_hbm.at[pl.ds(base0 + t * ch, ch)],
                                  dsts[b], sid[b]).wait()

        def issue_gc(t, b):
            pltpu.async_copy(z_hbm.at[srcs[b]], rows[b], sg[b])
            pltpu.async_copy(c_hbm.at[pl.ds(base0 + t * ch, ch)], cc[b], sc[b])

        def wait_gc(t, b):
            pltpu.make_async_copy(z_hbm.at[srcs[b]], rows[b], sg[b]).wait()
            pltpu.make_async_copy(c_hbm.at[pl.ds(base0 + t * ch, ch)],
                                  cc[b], sc[b]).wait()

        def compute(b):
            rows_v, cc_v = rows[b], cc[b]

            def erow(r, icarry):
                for j in range(h // 16):
                    sl = pl.ds(j * 16, 16)
                    rows_v[r, sl] = jnp.maximum(rows_v[r, sl] + cc_v[r, sl],
                                                0.0)
                return icarry
            lax.fori_loop(0, ch, erow, 0)

        def scatter(t, b):
            pltpu.async_copy(rows[b], agg_sh.at[dsts[b]], ss[b], add=True)

        def wait_s(t, b):
            pltpu.make_async_copy(rows[b], agg_sh.at[dsts[b]], ss[b]).wait()

        def step(t, b, first=False, last=False):
            nb = 1 - b
            if not last:
                issue_src(t + 1, nb)
            wait_gc(t, b)
            if not last:
                issue_dst(t + 1, nb)
                wait_src(t + 1, nb)
                issue_gc(t + 1, nb)
            compute(b)
            if not first:
                wait_dst(t, b)
            scatter(t, b)
            wait_s(t, b)

        pltpu.sync_copy(src_hbm.at[pl.ds(base0, ch)], srcs[0])
        pltpu.sync_copy(dst_hbm.at[pl.ds(base0, ch)], dsts[0])
        issue_gc(0, 0)
        step(0, 0, first=True)

        def pair(q, carry):
            t1 = 2 * q + 1
            step(t1, 1)
            step(t1 + 1, 0)
            return carry
        lax.fori_loop(0, (nchunk - 3) // 2, pair, 0)

        step(nchunk - 2, 1)
        step(nchunk - 1, 0, last=True)

        plsc.subcore_barrier()
        for k in range(rows_per_tile // ch):
            sl = pl.ds(row0 + k * ch, ch)
            pltpu.sync_copy(agg_sh.at[sl], out_hbm.at[ci, sl])

    return edge(z, c, src, dst)


def kernel(H, Xe, id_Xe, batch_idx,
           Wm0, bm0, Wu0, bu0,
           Wm1, bm1, Wu1, bu1,
           Wm2, bm2, Wu2, bu2,
           Wmlp, bmlp):
    n, d = H.shape
    e, de = Xe.shape
    hid = Wm0.shape[1]
    g = 64
    src = id_Xe[0]
    dst = id_Xe[1]

    c0, c1, c2 = _c_matmuls(Xe, Wm0[d:], Wm1[hid:], Wm2[hid:], block=2000)

    y = H
    for wm, bm, wu, bu, c in ((Wm0, bm0, Wu0, bu0, c0),
                              (Wm1, bm1, Wu1, bu1, c1),
                              (Wm2, bm2, Wu2, bu2, c2)):
        din = wm.shape[0] - de
        z = _z_matmul(y, wm[:din], bm, block=1000)
        parts = _edge_kernel(z, c, src, dst, n, e, hid)
        y = _update(parts[0], parts[1], y, wu[:hid], wu[hid:], bu, block=1000)

    return _pool(y, batch_idx, Wmlp, bmlp, g, block=1000)
